# Initial kernel scaffold; baseline (speedup 1.0000x reference)
import jax
import jax.numpy as jnp
from jax.experimental import pallas as pl


def kernel(patch, pos_table):
    raise NotImplementedError("write your pallas kernel here")


# speedup vs baseline: 1.4275x; 1.4275x over previous
import jax
import jax.numpy as jnp
from jax.experimental import pallas as pl


def _copy_body(table_ref, out_ref):
    out_ref[...] = table_ref[...]


def kernel(patch, pos_table):
    del patch
    return pl.pallas_call(
        _copy_body,
        out_shape=jax.ShapeDtypeStruct(pos_table.shape, pos_table.dtype),
    )(pos_table)
